# Initial kernel scaffold; baseline (speedup 1.0000x reference)
import jax
import jax.numpy as jnp
from jax.experimental import pallas as pl


def kernel(x, gate_w, w1, w2, w3):
    raise NotImplementedError("write your pallas kernel here")


# speedup vs baseline: 1.4702x; 1.4702x over previous
import functools

import jax
import jax.numpy as jnp
from jax.experimental import pallas as pl
from jax.experimental.pallas import tpu as pltpu


def _moe_dense_body(x_ref, gw_ref, w1_ref, w3_ref, w2_ref,
                    out_ref, aux_ref, c_ref, *, num_experts, ff_blocks):
    e = pl.program_id(0)
    f = pl.program_id(1)

    @pl.when((e == 0) & (f == 0))
    def _router():
        x = x_ref[...]
        logits = jax.lax.dot_general(
            x, gw_ref[...], (((1,), (1,)), ((), ())),
            preferred_element_type=jnp.float32)
        s = logits.shape[0]
        iota_e = jax.lax.broadcasted_iota(jnp.int32, (s, num_experts), 1)
        m1 = jnp.max(logits, axis=1, keepdims=True)
        big = jnp.int32(num_experts + 1)
        i1 = jnp.min(jnp.where(logits == m1, iota_e, big), axis=1, keepdims=True)
        masked = jnp.where(iota_e == i1, -jnp.inf, logits)
        m2 = jnp.max(masked, axis=1, keepdims=True)
        i2 = jnp.min(jnp.where(masked == m2, iota_e, big), axis=1, keepdims=True)
        p1 = jax.nn.sigmoid(m1 - m2)
        p2 = 1.0 - p1
        c_ref[...] = jnp.where(iota_e == i1, p1, 0.0) + jnp.where(iota_e == i2, p2, 0.0)
        probs = jax.nn.softmax(logits, axis=1)
        colsum = jnp.sum(probs, axis=0)
        aux_ref[0, 0] = num_experts * jnp.sum(colsum * colsum) / (s * s)
        out_ref[...] = jnp.zeros_like(out_ref)

    x = x_ref[...]
    a = jax.lax.dot_general(x, w1_ref[0], (((1,), (1,)), ((), ())),
                            preferred_element_type=jnp.float32)
    b = jax.lax.dot_general(x, w3_ref[0], (((1,), (1,)), ((), ())),
                            preferred_element_type=jnp.float32)
    h = (a * jax.nn.sigmoid(a)) * b
    y = jax.lax.dot_general(h, w2_ref[0], (((1,), (1,)), ((), ())),
                            preferred_element_type=jnp.float32)
    iota_e = jax.lax.broadcasted_iota(jnp.int32, c_ref.shape, 1)
    ce = jnp.sum(jnp.where(iota_e == e, c_ref[...], 0.0), axis=1, keepdims=True)
    out_ref[...] += y * ce


def kernel(x, gate_w, w1, w2, w3):
    batch, seq, hidden = x.shape
    num_experts, ff, _ = w1.shape
    s = batch * seq
    x_flat = x.reshape(s, hidden)

    fblk = min(ff, 1024)
    nf = ff // fblk

    out, aux = pl.pallas_call(
        functools.partial(_moe_dense_body, num_experts=num_experts, ff_blocks=nf),
        grid=(num_experts, nf),
        in_specs=[
            pl.BlockSpec((s, hidden), lambda e, f: (0, 0)),
            pl.BlockSpec((num_experts, hidden), lambda e, f: (0, 0)),
            pl.BlockSpec((1, fblk, hidden), lambda e, f: (e, f, 0)),
            pl.BlockSpec((1, fblk, hidden), lambda e, f: (e, f, 0)),
            pl.BlockSpec((1, hidden, fblk), lambda e, f: (e, 0, f)),
        ],
        out_specs=[
            pl.BlockSpec((s, hidden), lambda e, f: (0, 0)),
            pl.BlockSpec((1, 1), lambda e, f: (0, 0), memory_space=pltpu.SMEM),
        ],
        out_shape=[
            jax.ShapeDtypeStruct((s, hidden), jnp.float32),
            jax.ShapeDtypeStruct((1, 1), jnp.float32),
        ],
        scratch_shapes=[pltpu.VMEM((s, num_experts), jnp.float32)],
        compiler_params=pltpu.CompilerParams(
            dimension_semantics=("arbitrary", "arbitrary")),
    )(x_flat, gate_w, w1, w3, w2)

    return out.reshape(batch, seq, hidden), aux.reshape(())
